# baseline (speedup 1.0000x reference)
import functools

import jax
import jax.numpy as jnp
from jax import lax
from jax.experimental import pallas as pl
from jax.experimental.pallas import tpu as pltpu
from jax.experimental.pallas import tpu_sc as plsc

N = 10000
D = 128
E = 320000
NC = 2
NS = 16
NW = NC * NS
L = 16
CHUNK = 128
CPT = 80
E_PAD = CPT * NW * CHUNK
N_PAD = 10240
RPT = N_PAD // NS

_mesh = plsc.VectorSubcoreMesh(
    core_axis_name="c", subcore_axis_name="s", num_cores=NC, num_subcores=NS)


EPT = E_PAD // NW


@functools.partial(
    pl.kernel,
    out_type=(
        jax.ShapeDtypeStruct((NW, N_PAD), jnp.float32),
        jax.ShapeDtypeStruct((NW, N_PAD), jnp.float32),
    ),
    mesh=_mesh,
    compiler_params=pltpu.CompilerParams(needs_layout_passes=False),
    scratch_types=[
        pltpu.VMEM((EPT,), jnp.int32),
        pltpu.VMEM((EPT,), jnp.int32),
        pltpu.VMEM((N_PAD,), jnp.float32),
        pltpu.VMEM((N_PAD,), jnp.float32),
    ],
)
def _sc_degrees(src_hbm, dst_hbm, dout_hbm, din_hbm, sidx_v, didx_v, hs_v, hd_v):
    c = lax.axis_index("c")
    s = lax.axis_index("s")
    wid = c * NS + s
    zero16 = jnp.zeros((L,), jnp.float32)
    one16 = jnp.ones((L,), jnp.float32)

    def _zb(j, _):
        hs_v[pl.ds(j * L, L)] = zero16
        hd_v[pl.ds(j * L, L)] = zero16
        return 0
    lax.fori_loop(0, N_PAD // L, _zb, 0)

    base = wid * EPT
    pltpu.sync_copy(src_hbm.at[pl.ds(base, EPT)], sidx_v)
    pltpu.sync_copy(dst_hbm.at[pl.ds(base, EPT)], didx_v)

    def _eb(i, _):
        si = sidx_v[pl.ds(i * L, L)]
        di = didx_v[pl.ds(i * L, L)]
        plsc.addupdate_scatter(hs_v, [si], one16)
        plsc.addupdate_scatter(hd_v, [di], one16)
        return 0
    lax.fori_loop(0, EPT // L, _eb, 0)

    pltpu.sync_copy(hs_v, dout_hbm.at[wid])
    pltpu.sync_copy(hd_v, din_hbm.at[wid])


@functools.partial(
    pl.kernel,
    out_type=jax.ShapeDtypeStruct((NC, N_PAD, D), jnp.float32),
    mesh=_mesh,
    scratch_types=(
        [pltpu.VMEM((CHUNK,), jnp.int32)] * 2
        + [pltpu.VMEM((CHUNK,), jnp.int32)] * 4
        + [pltpu.VMEM((CHUNK, D), jnp.float32)] * 2
        + [pltpu.SemaphoreType.DMA] * 6
        + [pltpu.SemaphoreType.DMA] * 2
        + [pltpu.SemaphoreType.DMA] * 2
        + [pltpu.VMEM_SHARED((N_PAD, D), jnp.float32)]
    ),
)
def _sc_gs(y_hbm, src_hbm, dst_hbm, out_hbm,
           si0, si1, di0, di1, di2, di3, r0, r1,
           ss0, ss1, ds0, ds1, ds2, ds3, gs0, gs1,
           cs0, cs1, sh_agg):
    c = lax.axis_index("c")
    s = lax.axis_index("s")
    wid = c * NS + s
    zero16 = jnp.zeros((L,), jnp.float32)
    base = wid * CPT * CHUNK

    sidx = (si0, si1)
    didx = (di0, di1, di2, di3)
    rows = (r0, r1)
    ssem = (ss0, ss1)
    dsem = (ds0, ds1, ds2, ds3)
    gsem = (gs0, gs1)
    csem = (cs0, cs1)

    def fetch_idx(i, b2, b4):
        off = base + i * CHUNK
        pltpu.async_copy(src_hbm.at[pl.ds(off, CHUNK)], sidx[b2], ssem[b2])
        pltpu.async_copy(dst_hbm.at[pl.ds(off, CHUNK)], didx[b4], dsem[b4])

    def wait_idx(i, b2, b4):
        off = base + i * CHUNK
        pltpu.make_async_copy(src_hbm.at[pl.ds(off, CHUNK)], sidx[b2], ssem[b2]).wait()
        pltpu.make_async_copy(dst_hbm.at[pl.ds(off, CHUNK)], didx[b4], dsem[b4]).wait()

    def start_gather(b2):
        pltpu.async_copy(y_hbm.at[sidx[b2]], rows[b2], gsem[b2])

    def wait_gather(b2):
        pltpu.make_async_copy(y_hbm.at[sidx[b2]], rows[b2], gsem[b2]).wait()

    def start_scatter(b2, b4):
        pltpu.async_copy(rows[b2], sh_agg.at[didx[b4]], csem[b2], add=True)

    def wait_scatter(b2, b4):
        pltpu.make_async_copy(rows[b2], sh_agg.at[didx[b4]], csem[b2]).wait()

    def _zb(j, _):
        for l in range(D // L):
            r0[j, pl.ds(l * L, L)] = zero16
        return 0
    lax.fori_loop(0, CHUNK, _zb, 0)
    for k in range(RPT // CHUNK):
        pltpu.sync_copy(r0, sh_agg.at[pl.ds(s * RPT + k * CHUNK, CHUNK)])
    plsc.subcore_barrier()

    GP = CPT // 4
    fetch_idx(0, 0, 0)
    fetch_idx(1, 1, 1)
    wait_idx(0, 0, 0)
    start_gather(0)

    def _eb(p, _):
        g = p * 4
        for b in range(4):
            i = g + b
            b2 = b % 2
            o2 = (b + 1) % 2
            wait_gather(b2)
            start_scatter(b2, b)
            if b < 2:
                fetch_idx(i + 2, b2, (b + 2) % 4)
            else:
                @pl.when(p < GP - 1)
                def _():
                    fetch_idx(i + 2, b2, (b + 2) % 4)
            if b > 0:
                wait_scatter(o2, b - 1)
            else:
                @pl.when(p > 0)
                def _():
                    wait_scatter(o2, 3)
            if b < 3:
                wait_idx(i + 1, o2, (b + 1) % 4)
                start_gather(o2)
            else:
                @pl.when(p < GP - 1)
                def _():
                    wait_idx(i + 1, o2, 0)
                    start_gather(o2)
        return 0
    lax.fori_loop(0, GP, _eb, 0)
    wait_scatter(1, 3)
    plsc.subcore_barrier()

    def _wb(k, _):
        ro = s * RPT + k * CHUNK
        pltpu.sync_copy(sh_agg.at[pl.ds(ro, CHUNK)], r0)
        pltpu.sync_copy(r0, out_hbm.at[c, pl.ds(ro, CHUNK)])
        return 0
    lax.fori_loop(0, RPT // CHUNK, _wb, 0)


BR = 512
GRID = -(-N // BR)


def _norm_from(deg_ref):
    d = jnp.reshape(jnp.sum(deg_ref[...], axis=0), (BR, 1))
    return lax.rsqrt(jnp.maximum(d, 1.0))


def _tc_pre_body(x_ref, w_ref, dout_ref, o_ref):
    ns = _norm_from(dout_ref)
    y = jnp.dot(x_ref[...], w_ref[...], preferred_element_type=jnp.float32)
    o_ref[...] = y * ns


def _tc_mid_body(a_ref, dout_ref, din_ref, b_ref, w_ref, o_ref):
    agg = a_ref[0] + a_ref[1]
    nd = _norm_from(din_ref)
    ns = _norm_from(dout_ref)
    h = jnp.maximum(agg * nd + jnp.reshape(b_ref[...], (1, D)), 0.0)
    y = jnp.dot(h, w_ref[...], preferred_element_type=jnp.float32)
    o_ref[...] = y * ns


def _tc_post_body(a_ref, din_ref, b_ref, o_ref):
    agg = a_ref[0] + a_ref[1]
    nd = _norm_from(din_ref)
    o_ref[...] = agg * nd + jnp.reshape(b_ref[...], (1, D))


_row_spec = pl.BlockSpec((BR, D), lambda i: (i, 0))
_deg_spec = pl.BlockSpec((NW, BR), lambda i: (0, i))
_agg_spec = pl.BlockSpec((NC, BR, D), lambda i: (0, i, 0))
_w_spec = pl.BlockSpec((D, D), lambda i: (0, 0))
_b_spec = pl.BlockSpec((D,), lambda i: (0,))
_out_sds = jax.ShapeDtypeStruct((N, D), jnp.float32)

_tc_pre = pl.pallas_call(
    _tc_pre_body, grid=(GRID,),
    in_specs=[_row_spec, _w_spec, _deg_spec],
    out_specs=_row_spec, out_shape=_out_sds)

_tc_mid = pl.pallas_call(
    _tc_mid_body, grid=(GRID,),
    in_specs=[_agg_spec, _deg_spec, _deg_spec, _b_spec, _w_spec],
    out_specs=_row_spec, out_shape=_out_sds)

_tc_post = pl.pallas_call(
    _tc_post_body, grid=(GRID,),
    in_specs=[_agg_spec, _deg_spec, _b_spec],
    out_specs=_row_spec, out_shape=_out_sds)


def kernel(x, edge_index, W1, b1, W2, b2, W3, b3):
    src = edge_index[0].astype(jnp.int32)
    dst = edge_index[1].astype(jnp.int32)
    pad = E_PAD - E
    spread = jnp.arange(pad, dtype=jnp.int32)
    src_g = jnp.concatenate([src, spread % N])
    dummy = N + spread % (N_PAD - N)
    src_d = jnp.concatenate([src, dummy])
    dst_p = jnp.concatenate([dst, dummy])

    deg_out, deg_in = _sc_degrees(src_d, dst_p)

    y = _tc_pre(x, W1, deg_out)
    agg = _sc_gs(y, src_g, dst_p)
    y = _tc_mid(agg, deg_out, deg_in, b1, W2)
    agg = _sc_gs(y, src_g, dst_p)
    y = _tc_mid(agg, deg_out, deg_in, b2, W3)
    agg = _sc_gs(y, src_g, dst_p)
    return _tc_post(agg, deg_in, b3)


# speedup vs baseline: 1.0439x; 1.0439x over previous
import functools

import jax
import jax.numpy as jnp
from jax import lax
from jax.experimental import pallas as pl
from jax.experimental.pallas import tpu as pltpu
from jax.experimental.pallas import tpu_sc as plsc

N = 10000
D = 128
E = 320000
NC = 2
NS = 16
NW = NC * NS
L = 16
CHUNK = 128
CPT = 80
E_PAD = CPT * NW * CHUNK
N_PAD = 10240
RPT = N_PAD // NS

_mesh = plsc.VectorSubcoreMesh(
    core_axis_name="c", subcore_axis_name="s", num_cores=NC, num_subcores=NS)


EPT = E_PAD // NW


@functools.partial(
    pl.kernel,
    out_type=(
        jax.ShapeDtypeStruct((NW, N_PAD), jnp.float32),
        jax.ShapeDtypeStruct((NW, N_PAD), jnp.float32),
    ),
    mesh=_mesh,
    compiler_params=pltpu.CompilerParams(needs_layout_passes=False),
    scratch_types=[
        pltpu.VMEM((EPT,), jnp.int32),
        pltpu.VMEM((EPT,), jnp.int32),
        pltpu.VMEM((N_PAD,), jnp.float32),
        pltpu.VMEM((N_PAD,), jnp.float32),
    ],
)
def _sc_degrees(src_hbm, dst_hbm, dout_hbm, din_hbm, sidx_v, didx_v, hs_v, hd_v):
    c = lax.axis_index("c")
    s = lax.axis_index("s")
    wid = c * NS + s
    zero16 = jnp.zeros((L,), jnp.float32)
    one16 = jnp.ones((L,), jnp.float32)

    def _zb(j, _):
        hs_v[pl.ds(j * L, L)] = zero16
        hd_v[pl.ds(j * L, L)] = zero16
        return 0
    lax.fori_loop(0, N_PAD // L, _zb, 0)

    base = wid * EPT
    pltpu.sync_copy(src_hbm.at[pl.ds(base, EPT)], sidx_v)
    pltpu.sync_copy(dst_hbm.at[pl.ds(base, EPT)], didx_v)

    def _eb(i, _):
        si = sidx_v[pl.ds(i * L, L)]
        di = didx_v[pl.ds(i * L, L)]
        plsc.addupdate_scatter(hs_v, [si], one16)
        plsc.addupdate_scatter(hd_v, [di], one16)
        return 0
    lax.fori_loop(0, EPT // L, _eb, 0)

    pltpu.sync_copy(hs_v, dout_hbm.at[wid])
    pltpu.sync_copy(hd_v, din_hbm.at[wid])


@functools.partial(
    pl.kernel,
    out_type=jax.ShapeDtypeStruct((NC, N_PAD, D), jnp.float32),
    mesh=_mesh,
    scratch_types=(
        [pltpu.VMEM((CHUNK,), jnp.int32)] * 2
        + [pltpu.VMEM((CHUNK,), jnp.int32)] * 2
        + [pltpu.VMEM((CHUNK, D), jnp.float32)] * 2
        + [pltpu.SemaphoreType.DMA] * 6
        + [pltpu.VMEM_SHARED((N_PAD, D), jnp.float32)]
    ),
)
def _sc_gs(y_hbm, src_hbm, dst_hbm, out_hbm, si0, si1, di0, di1,
           r0, r1, ss0, ss1, ds0, ds1, gs0, gs1, sh_agg):
    c = lax.axis_index("c")
    s = lax.axis_index("s")
    wid = c * NS + s
    zero16 = jnp.zeros((L,), jnp.float32)
    base = wid * CPT * CHUNK

    sidx = (si0, si1)
    didx = (di0, di1)
    rows = (r0, r1)
    ssem = (ss0, ss1)
    dsem = (ds0, ds1)
    gsem = (gs0, gs1)

    def fetch_idx(i, b):
        off = base + i * CHUNK
        pltpu.async_copy(src_hbm.at[pl.ds(off, CHUNK)], sidx[b], ssem[b])
        pltpu.async_copy(dst_hbm.at[pl.ds(off, CHUNK)], didx[b], dsem[b])

    def wait_idx(i, b):
        off = base + i * CHUNK
        pltpu.make_async_copy(src_hbm.at[pl.ds(off, CHUNK)], sidx[b], ssem[b]).wait()
        pltpu.make_async_copy(dst_hbm.at[pl.ds(off, CHUNK)], didx[b], dsem[b]).wait()

    def start_gather(b):
        pltpu.async_copy(y_hbm.at[sidx[b]], rows[b], gsem[b])

    def wait_gather(b):
        pltpu.make_async_copy(y_hbm.at[sidx[b]], rows[b], gsem[b]).wait()

    def scatter(b):
        pltpu.sync_copy(rows[b], sh_agg.at[didx[b]], add=True)

    def _zb(j, _):
        for l in range(D // L):
            r0[j, pl.ds(l * L, L)] = zero16
        return 0
    lax.fori_loop(0, CHUNK, _zb, 0)
    for k in range(RPT // CHUNK):
        pltpu.sync_copy(r0, sh_agg.at[pl.ds(s * RPT + k * CHUNK, CHUNK)])
    plsc.subcore_barrier()

    fetch_idx(0, 0)
    fetch_idx(1, 1)
    wait_idx(0, 0)
    start_gather(0)

    def _eb(p, _):
        g = p * 2
        wait_idx(g + 1, 1)
        start_gather(1)
        wait_gather(0)
        scatter(0)
        @pl.when(p < CPT // 2 - 1)
        def _():
            fetch_idx(g + 2, 0)
        @pl.when(p < CPT // 2 - 1)
        def _():
            wait_idx(g + 2, 0)
            start_gather(0)
        wait_gather(1)
        scatter(1)
        @pl.when(p < CPT // 2 - 1)
        def _():
            fetch_idx(g + 3, 1)
        return 0
    lax.fori_loop(0, CPT // 2, _eb, 0)
    plsc.subcore_barrier()

    def _wb(k, _):
        ro = s * RPT + k * CHUNK
        pltpu.sync_copy(sh_agg.at[pl.ds(ro, CHUNK)], r0)
        pltpu.sync_copy(r0, out_hbm.at[c, pl.ds(ro, CHUNK)])
        return 0
    lax.fori_loop(0, RPT // CHUNK, _wb, 0)


BR = 512
GRID = -(-N // BR)


def _norm_from(deg_ref):
    d = jnp.reshape(jnp.sum(deg_ref[...], axis=0), (BR, 1))
    return lax.rsqrt(jnp.maximum(d, 1.0))


def _tc_pre_body(x_ref, w_ref, dout_ref, o_ref):
    ns = _norm_from(dout_ref)
    y = jnp.dot(x_ref[...], w_ref[...], preferred_element_type=jnp.float32)
    o_ref[...] = y * ns


def _tc_mid_body(a_ref, dout_ref, din_ref, b_ref, w_ref, o_ref):
    agg = a_ref[0] + a_ref[1]
    nd = _norm_from(din_ref)
    ns = _norm_from(dout_ref)
    h = jnp.maximum(agg * nd + jnp.reshape(b_ref[...], (1, D)), 0.0)
    y = jnp.dot(h, w_ref[...], preferred_element_type=jnp.float32)
    o_ref[...] = y * ns


def _tc_post_body(a_ref, din_ref, b_ref, o_ref):
    agg = a_ref[0] + a_ref[1]
    nd = _norm_from(din_ref)
    o_ref[...] = agg * nd + jnp.reshape(b_ref[...], (1, D))


_row_spec = pl.BlockSpec((BR, D), lambda i: (i, 0))
_deg_spec = pl.BlockSpec((NW, BR), lambda i: (0, i))
_agg_spec = pl.BlockSpec((NC, BR, D), lambda i: (0, i, 0))
_w_spec = pl.BlockSpec((D, D), lambda i: (0, 0))
_b_spec = pl.BlockSpec((D,), lambda i: (0,))
_out_sds = jax.ShapeDtypeStruct((N, D), jnp.float32)

_tc_pre = pl.pallas_call(
    _tc_pre_body, grid=(GRID,),
    in_specs=[_row_spec, _w_spec, _deg_spec],
    out_specs=_row_spec, out_shape=_out_sds)

_tc_mid = pl.pallas_call(
    _tc_mid_body, grid=(GRID,),
    in_specs=[_agg_spec, _deg_spec, _deg_spec, _b_spec, _w_spec],
    out_specs=_row_spec, out_shape=_out_sds)

_tc_post = pl.pallas_call(
    _tc_post_body, grid=(GRID,),
    in_specs=[_agg_spec, _deg_spec, _b_spec],
    out_specs=_row_spec, out_shape=_out_sds)


def kernel(x, edge_index, W1, b1, W2, b2, W3, b3):
    src = edge_index[0].astype(jnp.int32)
    dst = edge_index[1].astype(jnp.int32)
    pad = E_PAD - E
    spread = jnp.arange(pad, dtype=jnp.int32)
    src_g = jnp.concatenate([src, spread % N])
    dummy = N + spread % (N_PAD - N)
    src_d = jnp.concatenate([src, dummy])
    dst_p = jnp.concatenate([dst, dummy])

    deg_out, deg_in = _sc_degrees(src_d, dst_p)

    y = _tc_pre(x, W1, deg_out)
    agg = _sc_gs(y, src_g, dst_p)
    y = _tc_mid(agg, deg_out, deg_in, b1, W2)
    agg = _sc_gs(y, src_g, dst_p)
    y = _tc_mid(agg, deg_out, deg_in, b2, W3)
    agg = _sc_gs(y, src_g, dst_p)
    return _tc_post(agg, deg_in, b3)


# speedup vs baseline: 1.0504x; 1.0062x over previous
import functools

import jax
import jax.numpy as jnp
from jax import lax
from jax.experimental import pallas as pl
from jax.experimental.pallas import tpu as pltpu
from jax.experimental.pallas import tpu_sc as plsc

N = 10000
D = 128
E = 320000
NC = 2
NS = 16
NW = NC * NS
L = 16
CHUNK = 128
CPT = 80
E_PAD = CPT * NW * CHUNK
N_PAD = 10240
RPT = N_PAD // NS

_mesh = plsc.VectorSubcoreMesh(
    core_axis_name="c", subcore_axis_name="s", num_cores=NC, num_subcores=NS)


EPT = E_PAD // NW


@functools.partial(
    pl.kernel,
    out_type=(
        jax.ShapeDtypeStruct((NW, N_PAD), jnp.float32),
        jax.ShapeDtypeStruct((NW, N_PAD), jnp.float32),
    ),
    mesh=_mesh,
    compiler_params=pltpu.CompilerParams(needs_layout_passes=False),
    scratch_types=[
        pltpu.VMEM((EPT,), jnp.int32),
        pltpu.VMEM((EPT,), jnp.int32),
        pltpu.VMEM((N_PAD,), jnp.float32),
        pltpu.VMEM((N_PAD,), jnp.float32),
    ],
)
def _sc_degrees(src_hbm, dst_hbm, dout_hbm, din_hbm, sidx_v, didx_v, hs_v, hd_v):
    c = lax.axis_index("c")
    s = lax.axis_index("s")
    wid = c * NS + s
    zero16 = jnp.zeros((L,), jnp.float32)
    one16 = jnp.ones((L,), jnp.float32)

    def _zb(j, _):
        hs_v[pl.ds(j * L, L)] = zero16
        hd_v[pl.ds(j * L, L)] = zero16
        return 0
    lax.fori_loop(0, N_PAD // L, _zb, 0)

    base = wid * EPT
    pltpu.sync_copy(src_hbm.at[pl.ds(base, EPT)], sidx_v)
    pltpu.sync_copy(dst_hbm.at[pl.ds(base, EPT)], didx_v)

    def _eb(i, _):
        si = sidx_v[pl.ds(i * L, L)]
        di = didx_v[pl.ds(i * L, L)]
        plsc.addupdate_scatter(hs_v, [si], one16)
        plsc.addupdate_scatter(hd_v, [di], one16)
        return 0
    lax.fori_loop(0, EPT // L, _eb, 0)

    pltpu.sync_copy(hs_v, dout_hbm.at[wid])
    pltpu.sync_copy(hd_v, din_hbm.at[wid])


@functools.partial(
    pl.kernel,
    out_type=jax.ShapeDtypeStruct((NC, N_PAD, D), jnp.float32),
    mesh=_mesh,
    scratch_types=(
        [pltpu.VMEM((CHUNK,), jnp.int32)] * 2
        + [pltpu.VMEM((CHUNK,), jnp.int32)] * 2
        + [pltpu.VMEM((CHUNK, D), jnp.float32)] * 2
        + [pltpu.SemaphoreType.DMA] * 6
        + [pltpu.VMEM_SHARED((N_PAD, D), jnp.float32)]
    ),
)
def _sc_gs(y_hbm, src_hbm, dst_hbm, out_hbm, si0, si1, di0, di1,
           r0, r1, ss0, ss1, ds0, ds1, gs0, gs1, sh_agg):
    c = lax.axis_index("c")
    s = lax.axis_index("s")
    wid = c * NS + s
    zero16 = jnp.zeros((L,), jnp.float32)
    base = wid * CPT * CHUNK

    sidx = (si0, si1)
    didx = (di0, di1)
    rows = (r0, r1)
    ssem = (ss0, ss1)
    dsem = (ds0, ds1)
    gsem = (gs0, gs1)

    def fetch_idx(i, b):
        off = base + i * CHUNK
        pltpu.async_copy(src_hbm.at[pl.ds(off, CHUNK)], sidx[b], ssem[b])
        pltpu.async_copy(dst_hbm.at[pl.ds(off, CHUNK)], didx[b], dsem[b])

    def wait_idx(i, b):
        off = base + i * CHUNK
        pltpu.make_async_copy(src_hbm.at[pl.ds(off, CHUNK)], sidx[b], ssem[b]).wait()
        pltpu.make_async_copy(dst_hbm.at[pl.ds(off, CHUNK)], didx[b], dsem[b]).wait()

    def start_gather(b):
        pltpu.async_copy(y_hbm.at[sidx[b]], rows[b], gsem[b])

    def wait_gather(b):
        pltpu.make_async_copy(y_hbm.at[sidx[b]], rows[b], gsem[b]).wait()

    def scatter(b):
        pltpu.sync_copy(rows[b], sh_agg.at[didx[b]], add=True)

    def _zb(j, _):
        for l in range(D // L):
            r0[j, pl.ds(l * L, L)] = zero16
        return 0
    lax.fori_loop(0, CHUNK, _zb, 0)
    for k in range(RPT // CHUNK):
        pltpu.sync_copy(r0, sh_agg.at[pl.ds(s * RPT + k * CHUNK, CHUNK)])
    plsc.subcore_barrier()

    fetch_idx(0, 0)
    fetch_idx(1, 1)
    wait_idx(0, 0)
    start_gather(0)

    def _eb(p, _):
        g = p * 2
        wait_idx(g + 1, 1)
        start_gather(1)
        wait_gather(0)
        scatter(0)
        @pl.when(p < CPT // 2 - 1)
        def _():
            fetch_idx(g + 2, 0)
        @pl.when(p < CPT // 2 - 1)
        def _():
            wait_idx(g + 2, 0)
            start_gather(0)
        wait_gather(1)
        scatter(1)
        @pl.when(p < CPT // 2 - 1)
        def _():
            fetch_idx(g + 3, 1)
        return 0
    lax.fori_loop(0, CPT // 2, _eb, 0)
    plsc.subcore_barrier()

    def _wb(k, _):
        ro = s * RPT + k * CHUNK
        pltpu.sync_copy(sh_agg.at[pl.ds(ro, CHUNK)], out_hbm.at[c, pl.ds(ro, CHUNK)])
        return 0
    lax.fori_loop(0, RPT // CHUNK, _wb, 0)


BR = 512
GRID = -(-N // BR)


def _norm_from(deg_ref):
    d = jnp.reshape(jnp.sum(deg_ref[...], axis=0), (BR, 1))
    return lax.rsqrt(jnp.maximum(d, 1.0))


def _tc_pre_body(x_ref, w_ref, dout_ref, o_ref):
    ns = _norm_from(dout_ref)
    y = jnp.dot(x_ref[...], w_ref[...], preferred_element_type=jnp.float32)
    o_ref[...] = y * ns


def _tc_mid_body(a_ref, dout_ref, din_ref, b_ref, w_ref, o_ref):
    agg = a_ref[0] + a_ref[1]
    nd = _norm_from(din_ref)
    ns = _norm_from(dout_ref)
    h = jnp.maximum(agg * nd + jnp.reshape(b_ref[...], (1, D)), 0.0)
    y = jnp.dot(h, w_ref[...], preferred_element_type=jnp.float32)
    o_ref[...] = y * ns


def _tc_post_body(a_ref, din_ref, b_ref, o_ref):
    agg = a_ref[0] + a_ref[1]
    nd = _norm_from(din_ref)
    o_ref[...] = agg * nd + jnp.reshape(b_ref[...], (1, D))


_row_spec = pl.BlockSpec((BR, D), lambda i: (i, 0))
_deg_spec = pl.BlockSpec((NW, BR), lambda i: (0, i))
_agg_spec = pl.BlockSpec((NC, BR, D), lambda i: (0, i, 0))
_w_spec = pl.BlockSpec((D, D), lambda i: (0, 0))
_b_spec = pl.BlockSpec((D,), lambda i: (0,))
_out_sds = jax.ShapeDtypeStruct((N, D), jnp.float32)

_tc_pre = pl.pallas_call(
    _tc_pre_body, grid=(GRID,),
    in_specs=[_row_spec, _w_spec, _deg_spec],
    out_specs=_row_spec, out_shape=_out_sds)

_tc_mid = pl.pallas_call(
    _tc_mid_body, grid=(GRID,),
    in_specs=[_agg_spec, _deg_spec, _deg_spec, _b_spec, _w_spec],
    out_specs=_row_spec, out_shape=_out_sds)

_tc_post = pl.pallas_call(
    _tc_post_body, grid=(GRID,),
    in_specs=[_agg_spec, _deg_spec, _b_spec],
    out_specs=_row_spec, out_shape=_out_sds)


def kernel(x, edge_index, W1, b1, W2, b2, W3, b3):
    src = edge_index[0].astype(jnp.int32)
    dst = edge_index[1].astype(jnp.int32)
    pad = E_PAD - E
    spread = jnp.arange(pad, dtype=jnp.int32)
    src_g = jnp.concatenate([src, spread % N])
    dummy = N + spread % (N_PAD - N)
    src_d = jnp.concatenate([src, dummy])
    dst_p = jnp.concatenate([dst, dummy])

    deg_out, deg_in = _sc_degrees(src_d, dst_p)

    y = _tc_pre(x, W1, deg_out)
    agg = _sc_gs(y, src_g, dst_p)
    y = _tc_mid(agg, deg_out, deg_in, b1, W2)
    agg = _sc_gs(y, src_g, dst_p)
    y = _tc_mid(agg, deg_out, deg_in, b2, W3)
    agg = _sc_gs(y, src_g, dst_p)
    return _tc_post(agg, deg_in, b3)
